# Initial kernel scaffold; baseline (speedup 1.0000x reference)
import jax
import jax.numpy as jnp
from jax.experimental import pallas as pl


def kernel(logits, generated_tokens):
    raise NotImplementedError("write your pallas kernel here")


# speedup vs baseline: 2.9383x; 2.9383x over previous
import functools

import jax
import jax.numpy as jnp
from jax import lax
from jax.experimental import pallas as pl
from jax.experimental.pallas import tpu as pltpu
from jax.experimental.pallas import tpu_sc as plsc

B, S, V = 8, 256, 8192
WINDOW = 4
PENALTY = 0.3

NUM_CORES = 2
NUM_SUBCORES = 16
NUM_WORKERS = NUM_CORES * NUM_SUBCORES
ROWS = B * S
ROWS_PER_WORKER = ROWS // NUM_WORKERS
CHUNK = 4
NCHUNKS = ROWS_PER_WORKER // CHUNK


def _sc_body(logits_hbm, tok_hbm, out_hbm, tok_v, buf_a, buf_b,
             gsem_a, gsem_b, ssem_a, ssem_b):
    cid = lax.axis_index("c")
    sid = lax.axis_index("s")
    wid = sid * NUM_CORES + cid
    base = wid * ROWS_PER_WORKER
    batch = base // S
    s0 = base % S

    pltpu.sync_copy(tok_hbm.at[batch], tok_v)

    lane = lax.iota(jnp.int32, 16)
    zero16 = jnp.full((16,), 0, jnp.int32)
    ones16 = jnp.full((16,), 1, jnp.int32)
    win16 = jnp.full((16,), WINDOW, jnp.int32)
    r_idx = lax.shift_right_logical(lane, jnp.full((16,), 2, jnp.int32))
    d_idx = lax.bitwise_and(lane, jnp.full((16,), 3, jnp.int32))
    pen16 = jnp.full((16,), PENALTY, jnp.float32)

    def apply_penalty(buf, g):
        base_i = jnp.full((16,), s0 + g * CHUNK, jnp.int32)
        i_vec = lax.add(base_i, r_idx)
        valid = lax.ge(i_vec, win16)
        istart = lax.sub(i_vec, win16)
        col = plsc.load_gather(
            tok_v, [lax.max(lax.add(istart, d_idx), zero16)])
        cnt = None
        for k in range(WINDOW):
            pk = lax.max(lax.add(istart, jnp.full((16,), k, jnp.int32)),
                         zero16)
            wk = plsc.load_gather(tok_v, [pk])
            m = jnp.where(lax.eq(wk, col), ones16, zero16)
            cnt = m if cnt is None else lax.add(cnt, m)
        vals = plsc.load_gather(buf, [r_idx, col])
        newv = lax.sub(vals, lax.mul(pen16, cnt.astype(jnp.float32)))
        plsc.store_scatter(buf, [r_idx, col], newv, mask=valid)

    def rows_at(g):
        return logits_hbm.at[pl.ds(base + g * CHUNK, CHUNK)]

    def out_at(g):
        return out_hbm.at[pl.ds(base + g * CHUNK, CHUNK)]

    pltpu.async_copy(rows_at(0), buf_a, gsem_a)
    pltpu.async_copy(rows_at(1), buf_b, gsem_b)

    def process(buf, gsem, ssem, g, prefetch):
        pltpu.make_async_copy(rows_at(g), buf, gsem).wait()
        apply_penalty(buf, g)
        pltpu.async_copy(buf, out_at(g), ssem).wait()
        if prefetch:
            pltpu.async_copy(rows_at(g + 2), buf, gsem)

    def step(k, carry):
        g = 2 * k
        process(buf_a, gsem_a, ssem_a, g, True)
        process(buf_b, gsem_b, ssem_b, g + 1, True)
        return carry

    lax.fori_loop(0, NCHUNKS // 2 - 1, step, 0)
    process(buf_a, gsem_a, ssem_a, NCHUNKS - 2, False)
    process(buf_b, gsem_b, ssem_b, NCHUNKS - 1, False)


@jax.jit
def _coverage_sc(logits2d, tokens):
    mesh = plsc.VectorSubcoreMesh(core_axis_name="c", subcore_axis_name="s")
    return pl.kernel(
        _sc_body,
        out_type=jax.ShapeDtypeStruct((ROWS, V), jnp.float32),
        mesh=mesh,
        compiler_params=pltpu.CompilerParams(needs_layout_passes=False),
        scratch_types=[
            pltpu.VMEM((S,), jnp.int32),
            pltpu.VMEM((CHUNK, V), jnp.float32),
            pltpu.VMEM((CHUNK, V), jnp.float32),
            pltpu.SemaphoreType.DMA,
            pltpu.SemaphoreType.DMA,
            pltpu.SemaphoreType.DMA,
            pltpu.SemaphoreType.DMA,
        ],
    )(logits2d, tokens)


def kernel(logits, generated_tokens):
    out = _coverage_sc(logits.reshape(ROWS, V), generated_tokens)
    return out.reshape(B, S, V)


# speedup vs baseline: 2.9559x; 1.0060x over previous
import functools

import jax
import jax.numpy as jnp
from jax import lax
from jax.experimental import pallas as pl
from jax.experimental.pallas import tpu as pltpu
from jax.experimental.pallas import tpu_sc as plsc

B, S, V = 8, 256, 8192
WINDOW = 4
PENALTY = 0.3

NUM_CORES = 2
NUM_SUBCORES = 16
NUM_WORKERS = NUM_CORES * NUM_SUBCORES
ROWS = B * S
ROWS_PER_WORKER = ROWS // NUM_WORKERS
CHUNK = 4
NCHUNKS = ROWS_PER_WORKER // CHUNK


def _sc_body(logits_hbm, tok_hbm, out_hbm, tok_v, buf_a, buf_b, buf_c,
             gsem_a, gsem_b, gsem_c, ssem_a, ssem_b, ssem_c):
    cid = lax.axis_index("c")
    sid = lax.axis_index("s")
    wid = sid * NUM_CORES + cid
    base = wid * ROWS_PER_WORKER
    batch = base // S
    s0 = base % S

    pltpu.sync_copy(tok_hbm.at[batch], tok_v)

    lane = lax.iota(jnp.int32, 16)
    zero16 = jnp.full((16,), 0, jnp.int32)
    ones16 = jnp.full((16,), 1, jnp.int32)
    win16 = jnp.full((16,), WINDOW, jnp.int32)
    r_idx = lax.shift_right_logical(lane, jnp.full((16,), 2, jnp.int32))
    d_idx = lax.bitwise_and(lane, jnp.full((16,), 3, jnp.int32))
    pen16 = jnp.full((16,), PENALTY, jnp.float32)

    def apply_penalty(buf, g):
        base_i = jnp.full((16,), s0 + g * CHUNK, jnp.int32)
        i_vec = lax.add(base_i, r_idx)
        valid = lax.ge(i_vec, win16)
        istart = lax.sub(i_vec, win16)
        col = plsc.load_gather(
            tok_v, [lax.max(lax.add(istart, d_idx), zero16)])
        cnt = None
        for k in range(WINDOW):
            pk = lax.max(lax.add(istart, jnp.full((16,), k, jnp.int32)),
                         zero16)
            wk = plsc.load_gather(tok_v, [pk])
            m = jnp.where(lax.eq(wk, col), ones16, zero16)
            cnt = m if cnt is None else lax.add(cnt, m)
        vals = plsc.load_gather(buf, [r_idx, col])
        newv = lax.sub(vals, lax.mul(pen16, cnt.astype(jnp.float32)))
        plsc.store_scatter(buf, [r_idx, col], newv, mask=valid)

    def rows_at(g):
        return logits_hbm.at[pl.ds(base + g * CHUNK, CHUNK)]

    def out_at(g):
        return out_hbm.at[pl.ds(base + g * CHUNK, CHUNK)]

    bufs = (buf_a, buf_b, buf_c)
    gsems = (gsem_a, gsem_b, gsem_c)
    ssems = (ssem_a, ssem_b, ssem_c)

    for g in range(3):
        pltpu.async_copy(rows_at(g), bufs[g], gsems[g])
    for g in range(NCHUNKS):
        b = g % 3
        if 1 <= g <= NCHUNKS - 3:
            pb = (g - 1) % 3
            pltpu.make_async_copy(bufs[pb], out_at(g - 1), ssems[pb]).wait()
            pltpu.async_copy(rows_at(g + 2), bufs[pb], gsems[pb])
        pltpu.make_async_copy(rows_at(g), bufs[b], gsems[b]).wait()
        apply_penalty(bufs[b], g)
        pltpu.async_copy(bufs[b], out_at(g), ssems[b])
    for g in range(NCHUNKS - 3, NCHUNKS):
        b = g % 3
        pltpu.make_async_copy(bufs[b], out_at(g), ssems[b]).wait()


@jax.jit
def _coverage_sc(logits2d, tokens):
    mesh = plsc.VectorSubcoreMesh(core_axis_name="c", subcore_axis_name="s")
    return pl.kernel(
        _sc_body,
        out_type=jax.ShapeDtypeStruct((ROWS, V), jnp.float32),
        mesh=mesh,
        compiler_params=pltpu.CompilerParams(needs_layout_passes=False),
        scratch_types=(
            [pltpu.VMEM((S,), jnp.int32)]
            + [pltpu.VMEM((CHUNK, V), jnp.float32)] * 3
            + [pltpu.SemaphoreType.DMA] * 6
        ),
    )(logits2d, tokens)


def kernel(logits, generated_tokens):
    out = _coverage_sc(logits.reshape(ROWS, V), generated_tokens)
    return out.reshape(B, S, V)


# speedup vs baseline: 2.9570x; 1.0004x over previous
import functools

import jax
import jax.numpy as jnp
from jax import lax
from jax.experimental import pallas as pl
from jax.experimental.pallas import tpu as pltpu
from jax.experimental.pallas import tpu_sc as plsc

B, S, V = 8, 256, 8192
WINDOW = 4
PENALTY = 0.3

NUM_CORES = 2
NUM_SUBCORES = 16
NUM_WORKERS = NUM_CORES * NUM_SUBCORES
ROWS = B * S
ROWS_PER_WORKER = ROWS // NUM_WORKERS
CHUNK = 4
NCHUNKS = ROWS_PER_WORKER // CHUNK


def _sc_body(logits_hbm, tok_hbm, out_hbm, tok_v, buf_a, buf_b, buf_c,
             gsem_a, gsem_b, gsem_c, ssem_a, ssem_b, ssem_c):
    cid = lax.axis_index("c")
    sid = lax.axis_index("s")
    wid = sid * NUM_CORES + cid
    base = wid * ROWS_PER_WORKER
    batch = base // S
    s0 = base % S

    pltpu.sync_copy(tok_hbm.at[batch], tok_v)

    lane = lax.iota(jnp.int32, 16)
    zero16 = jnp.full((16,), 0, jnp.int32)
    ones16 = jnp.full((16,), 1, jnp.int32)
    win16 = jnp.full((16,), WINDOW, jnp.int32)
    r_idx = lax.shift_right_logical(lane, jnp.full((16,), 2, jnp.int32))
    d_idx = lax.bitwise_and(lane, jnp.full((16,), 3, jnp.int32))
    pen16 = jnp.full((16,), PENALTY, jnp.float32)

    def apply_penalty(buf, g):
        base_i = jnp.full((16,), s0 + g * CHUNK, jnp.int32)
        i_vec = lax.add(base_i, r_idx)
        valid = lax.ge(i_vec, win16)
        istart = lax.sub(i_vec, win16)
        col = plsc.load_gather(
            tok_v, [lax.max(lax.add(istart, d_idx), zero16)])
        cnt = None
        for k in range(WINDOW):
            pk = lax.max(lax.add(istart, jnp.full((16,), k, jnp.int32)),
                         zero16)
            wk = plsc.load_gather(tok_v, [pk])
            m = jnp.where(lax.eq(wk, col), ones16, zero16)
            cnt = m if cnt is None else lax.add(cnt, m)
        vals = plsc.load_gather(buf, [r_idx, col])
        newv = lax.sub(vals, lax.mul(pen16, cnt.astype(jnp.float32)))
        plsc.store_scatter(buf, [r_idx, col], newv, mask=valid)

    def rows_at(g):
        return logits_hbm.at[pl.ds(base + g * CHUNK, CHUNK)]

    def out_at(g):
        return out_hbm.at[pl.ds(base + g * CHUNK, CHUNK)]

    bufs = (buf_a, buf_b, buf_c)
    gsems = (gsem_a, gsem_b, gsem_c)
    ssems = (ssem_a, ssem_b, ssem_c)

    for g in range(3):
        pltpu.async_copy(rows_at(g), bufs[g], gsems[g])
    for g in range(NCHUNKS):
        b = g % 3
        if 1 <= g <= NCHUNKS - 3:
            pb = (g - 1) % 3
            pltpu.make_async_copy(bufs[pb], out_at(g - 1), ssems[pb]).wait()
            pltpu.async_copy(rows_at(g + 2), bufs[pb], gsems[pb])
        pltpu.make_async_copy(rows_at(g), bufs[b], gsems[b]).wait()
        apply_penalty(bufs[b], g)
        pltpu.async_copy(bufs[b], out_at(g), ssems[b])
    for g in range(NCHUNKS - 3, NCHUNKS):
        b = g % 3
        pltpu.make_async_copy(bufs[b], out_at(g), ssems[b]).wait()


@jax.jit
def _coverage_sc(logits2d, tokens):
    mesh = plsc.VectorSubcoreMesh(core_axis_name="c", subcore_axis_name="s")
    return pl.kernel(
        _sc_body,
        out_type=jax.ShapeDtypeStruct((ROWS, V), jnp.float32),
        mesh=mesh,
        compiler_params=pltpu.CompilerParams(
            needs_layout_passes=False,
            disable_bounds_checks=True,
            disable_semaphore_checks=True,
        ),
        scratch_types=(
            [pltpu.VMEM((S,), jnp.int32)]
            + [pltpu.VMEM((CHUNK, V), jnp.float32)] * 3
            + [pltpu.SemaphoreType.DMA] * 6
        ),
    )(logits2d, tokens)


def kernel(logits, generated_tokens):
    out = _coverage_sc(logits.reshape(ROWS, V), generated_tokens)
    return out.reshape(B, S, V)
